# Initial kernel scaffold; baseline (speedup 1.0000x reference)
import jax
import jax.numpy as jnp
from jax.experimental import pallas as pl


def kernel(x, freq_emb, gate_w, freq_gate_w, p0, p1, p2, qw, qdw, kvw, kvdw, lnw, lnb, pow_w, pob):
    raise NotImplementedError("write your pallas kernel here")


# speedup vs baseline: 17.0235x; 17.0235x over previous
import functools

import numpy as np
import jax
import jax.numpy as jnp
from jax.experimental import pallas as pl
from jax.experimental.pallas import tpu as pltpu

B, C, H, W = 4, 64, 128, 128
E, K, R, FD, P = 8, 2, 16, 64, 8
HW = H * W
NP = H // P


def _dft_mats():
    idx = np.arange(P)
    F = np.exp(-2j * np.pi * np.outer(idx, idx) / P)
    M = np.kron(np.eye(NP), F)
    N = np.kron(np.eye(NP), np.conj(F) / P)
    f32 = np.float32
    return (jnp.asarray(M.real.astype(f32)), jnp.asarray(M.imag.astype(f32)),
            jnp.asarray(N.real.astype(f32)), jnp.asarray(N.imag.astype(f32)))


def _routing_body(x_ref, gwt_ref, fgwt_ref, fe_ref, noise_ref, idx_ref, val_ref):
    pooled = jnp.mean(x_ref[...], axis=(2, 3))
    logits = (jnp.dot(pooled, gwt_ref[...], preferred_element_type=jnp.float32)
              + jnp.dot(fe_ref[...], fgwt_ref[...], preferred_element_type=jnp.float32)
              + noise_ref[...])
    m = jnp.max(logits, axis=1, keepdims=True)
    ex = jnp.exp(logits - m)
    scores = ex / jnp.sum(ex, axis=1, keepdims=True)
    iota = jax.lax.broadcasted_iota(jnp.int32, (B, E), 1)
    m1 = jnp.max(scores, axis=1, keepdims=True)
    a1 = jnp.min(jnp.where(scores >= m1, iota, E), axis=1, keepdims=True)
    masked = jnp.where(iota == a1, -jnp.inf, scores)
    m2 = jnp.max(masked, axis=1, keepdims=True)
    a2 = jnp.min(jnp.where(masked >= m2, iota, E), axis=1, keepdims=True)
    idx2 = jnp.concatenate([a1, a2], axis=1)
    val2 = jnp.concatenate([m1, m2], axis=1)
    zi = jnp.zeros((B, 128 - K), jnp.int32)
    zf = jnp.zeros((B, 128 - K), jnp.float32)
    idx_row = jnp.concatenate([idx2, zi], axis=1)
    val_row = jnp.concatenate([val2, zf], axis=1)
    idx_ref[...] = jnp.concatenate([idx_row, jnp.zeros((8 - B, 128), jnp.int32)], axis=0)
    val_ref[...] = jnp.concatenate([val_row, jnp.zeros((8 - B, 128), jnp.float32)], axis=0)


NSTRIP = 4
S = H // NSTRIP


def _strip_pad(full, r0, r1, p):
    c = full.shape[0]
    top = max(r0 - p, 0)
    bot = min(r1 + p, H)
    parts = []
    if top - (r0 - p) > 0:
        parts.append(jnp.zeros((c, top - (r0 - p), W), full.dtype))
    parts.append(full[:, top:bot])
    if (r1 + p) - bot > 0:
        parts.append(jnp.zeros((c, (r1 + p) - bot, W), full.dtype))
    ph = jnp.concatenate(parts, axis=1) if len(parts) > 1 else parts[0]
    zc = jnp.zeros((c, (r1 - r0) + 2 * p, p), full.dtype)
    return jnp.concatenate([zc, ph, zc], axis=2)


def _dwconv_strip(full, w, p, r0, r1):
    zp = _strip_pad(full, r0, r1, p)
    s = r1 - r0
    kdim = 2 * p + 1
    acc = None
    for i in range(kdim):
        for j in range(kdim):
            wt = w[i * kdim + j][:, :, None]
            t = wt * zp[:, i:i + s, j:j + W]
            acc = t if acc is None else acc + t
    return acc


def _rmul(x3, a):
    c, m, n = x3.shape
    return jnp.dot(x3.reshape(c * m, n), a,
                   preferred_element_type=jnp.float32).reshape(c, m, n)


def _t2(x3):
    return jnp.swapaxes(x3, 1, 2)


def _expert_body(idx_ref, x_ref, val_ref, p0_ref, p1_ref, p2_ref, qw_ref,
                 qdw_ref, kvw_ref, kvdw_ref, lnw_ref, lnb_ref, pow_ref,
                 pob_ref, mr_ref, mi_ref, nr_ref, ni_ref, out_ref):
    b = pl.program_id(0)
    k = pl.program_id(1)
    gate = val_ref[b * K + k]

    x = x_ref[0]
    xm = x.reshape(C, HW)
    p0e = p0_ref[0]
    h0 = jnp.dot(p0e, xm, preferred_element_type=jnp.float32)

    q1 = jnp.dot(qw_ref[0], h0,
                 preferred_element_type=jnp.float32).reshape(R, H, W)
    kv1 = jnp.dot(kvw_ref[0], h0,
                  preferred_element_type=jnp.float32).reshape(2 * R, H, W)

    mr, mi, nr, ni = mr_ref[...], mi_ref[...], nr_ref[...], ni_ref[...]
    mrs, mis = mr[:S, :S], mi[:S, :S]
    nrs, nis = nr[:S, :S], ni[:S, :S]
    lnw = lnw_ref[0][:, :, None]
    lnb = lnb_ref[0][:, :, None]

    def fwd(z):
        ar = _rmul(z, mr)
        ai = _rmul(z, mi)
        atr, ati = _t2(ar), _t2(ai)
        return (_rmul(atr, mrs) - _rmul(ati, mis),
                _rmul(atr, mis) + _rmul(ati, mrs))

    for si in range(NSTRIP):
        r0, r1 = si * S, (si + 1) * S
        q = _dwconv_strip(q1, qdw_ref[0], 1, r0, r1)
        kv = _dwconv_strip(kv1, kvdw_ref[0], 3, r0, r1)
        k_ = kv[:R]
        v = kv[R:]

        qfr, qfi = fwd(q)
        kfr, kfi = fwd(k_)
        pr = qfr * kfr - qfi * kfi
        pi = qfr * kfi + qfi * kfr
        br = _rmul(pr, nrs) - _rmul(pi, nis)
        bi = _rmul(pr, nis) + _rmul(pi, nrs)
        attn = _rmul(_t2(br), nr) - _rmul(_t2(bi), ni)

        mu = jnp.mean(attn, axis=0, keepdims=True)
        var = jnp.mean((attn - mu) ** 2, axis=0, keepdims=True)
        normed = (attn - mu) * jax.lax.rsqrt(var + 1e-5) * lnw + lnb
        outa = normed * v

        po = jnp.dot(pow_ref[0], outa.reshape(R, S * W),
                     preferred_element_type=jnp.float32)
        po = po + pob_ref[0]

        xs = x[:, r0:r1]
        s_ = jnp.dot(p1_ref[0], xs.reshape(C, S * W),
                     preferred_element_type=jnp.float32)
        hh = po * (s_ * jax.nn.sigmoid(s_))

        contrib = jnp.dot(p2_ref[0], hh,
                          preferred_element_type=jnp.float32)
        res = gate * (contrib.reshape(C, S, W) + xs)

        @pl.when(k == 0)
        def _(res=res, r0=r0, r1=r1):
            out_ref[0, :, r0:r1, :] = res

        @pl.when(k != 0)
        def _(res=res, r0=r0, r1=r1):
            out_ref[0, :, r0:r1, :] = out_ref[0, :, r0:r1, :] + res


@jax.jit
def kernel(x, freq_emb, gate_w, freq_gate_w, p0, p1, p2, qw, qdw, kvw, kvdw,
           lnw, lnb, pow_w, pob):
    noise = jax.random.normal(jax.random.key(42), (B, E), jnp.float32) * (1.0 / E)
    idx_buf, val_buf = pl.pallas_call(
        _routing_body,
        out_shape=(jax.ShapeDtypeStruct((8, 128), jnp.int32),
                   jax.ShapeDtypeStruct((8, 128), jnp.float32)),
    )(x, gate_w.T, freq_gate_w.T, freq_emb, noise)

    idx_flat = idx_buf[:B, :K].reshape(B * K)
    val_flat = val_buf[:B, :K].reshape(B * K)

    mr, mi, nr, ni = _dft_mats()

    p0r = p0.reshape(E, R, C)
    p1r = p1.reshape(E, R, C)
    p2r = p2.reshape(E, C, R)
    qwr = qw.reshape(E, R, R)
    qdwr = qdw.reshape(E, R, 9).transpose(0, 2, 1).reshape(E, 9, R, 1)
    kvwr = kvw.reshape(E, 2 * R, R)
    kvdwr = kvdw.reshape(E, 2 * R, 49).transpose(0, 2, 1).reshape(E, 49, 2 * R, 1)
    lnwr = lnw.reshape(E, R, 1)
    lnbr = lnb.reshape(E, R, 1)
    powr = pow_w.reshape(E, R, R)
    pobr = pob.reshape(E, R, 1)

    def em(b, k, idx_ref):
        return (idx_ref[b * K + k], 0, 0)

    def em4(b, k, idx_ref):
        return (idx_ref[b * K + k], 0, 0, 0)

    def xb(b, k, idx_ref):
        return (b, 0, 0, 0)

    def zz(b, k, idx_ref):
        return (0, 0)

    grid_spec = pltpu.PrefetchScalarGridSpec(
        num_scalar_prefetch=1,
        grid=(B, K),
        in_specs=[
            pl.BlockSpec((1, C, H, W), xb),
            pl.BlockSpec(memory_space=pltpu.SMEM),
            pl.BlockSpec((1, R, C), em),
            pl.BlockSpec((1, R, C), em),
            pl.BlockSpec((1, C, R), em),
            pl.BlockSpec((1, R, R), em),
            pl.BlockSpec((1, 9, R, 1), em4),
            pl.BlockSpec((1, 2 * R, R), em),
            pl.BlockSpec((1, 49, 2 * R, 1), em4),
            pl.BlockSpec((1, R, 1), em),
            pl.BlockSpec((1, R, 1), em),
            pl.BlockSpec((1, R, R), em),
            pl.BlockSpec((1, R, 1), em),
            pl.BlockSpec((H, W), zz),
            pl.BlockSpec((H, W), zz),
            pl.BlockSpec((H, W), zz),
            pl.BlockSpec((H, W), zz),
        ],
        out_specs=pl.BlockSpec((1, C, H, W), xb),
    )

    out = pl.pallas_call(
        _expert_body,
        grid_spec=grid_spec,
        out_shape=jax.ShapeDtypeStruct((B, C, H, W), jnp.float32),
        compiler_params=pltpu.CompilerParams(
            dimension_semantics=("arbitrary", "arbitrary")),
    )(idx_flat, x, val_flat, p0r, p1r, p2r, qwr, qdwr, kvwr, kvdwr,
      lnwr, lnbr, powr, pobr, mr, mi, nr, ni)
    return out


# speedup vs baseline: 17.0300x; 1.0004x over previous
import functools

import numpy as np
import jax
import jax.numpy as jnp
from jax.experimental import pallas as pl
from jax.experimental.pallas import tpu as pltpu

B, C, H, W = 4, 64, 128, 128
E, K, R, FD, P = 8, 2, 16, 64, 8
HW = H * W
NP = H // P


def _dft_mats():
    idx = np.arange(P)
    F = np.exp(-2j * np.pi * np.outer(idx, idx) / P)
    M = np.kron(np.eye(NP), F)
    N = np.kron(np.eye(NP), np.conj(F) / P)
    f32 = np.float32
    return (jnp.asarray(M.real.astype(f32)), jnp.asarray(M.imag.astype(f32)),
            jnp.asarray(N.real.astype(f32)), jnp.asarray(N.imag.astype(f32)))


def _routing_body(x_ref, gwt_ref, fgwt_ref, fe_ref, noise_ref, idx_ref, val_ref):
    pooled = jnp.mean(x_ref[...], axis=(2, 3))
    logits = (jnp.dot(pooled, gwt_ref[...], preferred_element_type=jnp.float32)
              + jnp.dot(fe_ref[...], fgwt_ref[...], preferred_element_type=jnp.float32)
              + noise_ref[...])
    m = jnp.max(logits, axis=1, keepdims=True)
    ex = jnp.exp(logits - m)
    scores = ex / jnp.sum(ex, axis=1, keepdims=True)
    iota = jax.lax.broadcasted_iota(jnp.int32, (B, E), 1)
    m1 = jnp.max(scores, axis=1, keepdims=True)
    a1 = jnp.min(jnp.where(scores >= m1, iota, E), axis=1, keepdims=True)
    masked = jnp.where(iota == a1, -jnp.inf, scores)
    m2 = jnp.max(masked, axis=1, keepdims=True)
    a2 = jnp.min(jnp.where(masked >= m2, iota, E), axis=1, keepdims=True)
    idx2 = jnp.concatenate([a1, a2], axis=1)
    val2 = jnp.concatenate([m1, m2], axis=1)
    zi = jnp.zeros((B, 128 - K), jnp.int32)
    zf = jnp.zeros((B, 128 - K), jnp.float32)
    idx_row = jnp.concatenate([idx2, zi], axis=1)
    val_row = jnp.concatenate([val2, zf], axis=1)
    idx_ref[...] = jnp.concatenate([idx_row, jnp.zeros((8 - B, 128), jnp.int32)], axis=0)
    val_ref[...] = jnp.concatenate([val_row, jnp.zeros((8 - B, 128), jnp.float32)], axis=0)


NSTRIP = 4
S = H // NSTRIP


def _strip_pad(full, r0, r1, p):
    c = full.shape[0]
    top = max(r0 - p, 0)
    bot = min(r1 + p, H)
    parts = []
    if top - (r0 - p) > 0:
        parts.append(jnp.zeros((c, top - (r0 - p), W), full.dtype))
    parts.append(full[:, top:bot])
    if (r1 + p) - bot > 0:
        parts.append(jnp.zeros((c, (r1 + p) - bot, W), full.dtype))
    ph = jnp.concatenate(parts, axis=1) if len(parts) > 1 else parts[0]
    zc = jnp.zeros((c, (r1 - r0) + 2 * p, p), full.dtype)
    return jnp.concatenate([zc, ph, zc], axis=2)


def _dwconv_strip(full, w, p, r0, r1):
    zp = _strip_pad(full, r0, r1, p)
    s = r1 - r0
    kdim = 2 * p + 1
    acc = None
    for i in range(kdim):
        for j in range(kdim):
            wt = w[i * kdim + j][:, :, None]
            t = wt * zp[:, i:i + s, j:j + W]
            acc = t if acc is None else acc + t
    return acc


def _rmul(x3, a):
    c, m, n = x3.shape
    return jnp.dot(x3.reshape(c * m, n), a,
                   preferred_element_type=jnp.float32).reshape(c, m, n)


def _t2(x3):
    return jnp.swapaxes(x3, 1, 2)


def _expert_body(idx_ref, x_ref, val_ref, p0_ref, p1_ref, p2_ref, qw_ref,
                 qdw_ref, kvw_ref, kvdw_ref, lnw_ref, lnb_ref, pow_ref,
                 pob_ref, mr_ref, mi_ref, nr_ref, ni_ref, out_ref):
    b = pl.program_id(0)
    k = pl.program_id(1)
    gate = val_ref[b * K + k]

    x = x_ref[0]
    xm = x.reshape(C, HW)
    p0e = p0_ref[0]
    h0 = jnp.dot(p0e, xm, preferred_element_type=jnp.float32)

    q1 = jnp.dot(qw_ref[0], h0,
                 preferred_element_type=jnp.float32).reshape(R, H, W)
    kv1 = jnp.dot(kvw_ref[0], h0,
                  preferred_element_type=jnp.float32).reshape(2 * R, H, W)

    mr, mi, nr, ni = mr_ref[...], mi_ref[...], nr_ref[...], ni_ref[...]
    mrs, mis = mr[:S, :S], mi[:S, :S]
    nrs, nis = nr[:S, :S], ni[:S, :S]
    lnw = lnw_ref[0][:, :, None]
    lnb = lnb_ref[0][:, :, None]

    def fwd(z):
        ar = _rmul(z, mr)
        ai = _rmul(z, mi)
        atr, ati = _t2(ar), _t2(ai)
        return (_rmul(atr, mrs) - _rmul(ati, mis),
                _rmul(atr, mis) + _rmul(ati, mrs))

    for si in range(NSTRIP):
        r0, r1 = si * S, (si + 1) * S
        q = _dwconv_strip(q1, qdw_ref[0], 1, r0, r1)
        kv = _dwconv_strip(kv1, kvdw_ref[0], 3, r0, r1)
        k_ = kv[:R]
        v = kv[R:]

        qfr, qfi = fwd(q)
        kfr, kfi = fwd(k_)
        pr = qfr * kfr - qfi * kfi
        pi = qfr * kfi + qfi * kfr
        br = _rmul(pr, nrs) - _rmul(pi, nis)
        bi = _rmul(pr, nis) + _rmul(pi, nrs)
        attn = _rmul(_t2(br), nr) - _rmul(_t2(bi), ni)

        mu = jnp.mean(attn, axis=0, keepdims=True)
        var = jnp.mean((attn - mu) ** 2, axis=0, keepdims=True)
        normed = (attn - mu) * jax.lax.rsqrt(var + 1e-5) * lnw + lnb
        outa = normed * v

        po = jnp.dot(pow_ref[0], outa.reshape(R, S * W),
                     preferred_element_type=jnp.float32)
        po = po + pob_ref[0]

        xs = x[:, r0:r1]
        s_ = jnp.dot(p1_ref[0], xs.reshape(C, S * W),
                     preferred_element_type=jnp.float32)
        hh = po * (s_ * jax.nn.sigmoid(s_))

        contrib = jnp.dot(p2_ref[0], hh,
                          preferred_element_type=jnp.float32)
        res = gate * (contrib.reshape(C, S, W) + xs)

        @pl.when(k == 0)
        def _(res=res, r0=r0, r1=r1):
            out_ref[0, :, r0:r1, :] = res

        @pl.when(k != 0)
        def _(res=res, r0=r0, r1=r1):
            out_ref[0, :, r0:r1, :] = out_ref[0, :, r0:r1, :] + res


@jax.jit
def kernel(x, freq_emb, gate_w, freq_gate_w, p0, p1, p2, qw, qdw, kvw, kvdw,
           lnw, lnb, pow_w, pob):
    noise = jax.random.normal(jax.random.key(42), (B, E), jnp.float32) * (1.0 / E)
    idx_buf, val_buf = pl.pallas_call(
        _routing_body,
        out_shape=(jax.ShapeDtypeStruct((8, 128), jnp.int32),
                   jax.ShapeDtypeStruct((8, 128), jnp.float32)),
    )(x, gate_w.T, freq_gate_w.T, freq_emb, noise)

    idx_flat = idx_buf[:B, :K].reshape(B * K)
    val_flat = val_buf[:B, :K].reshape(B * K)

    mr, mi, nr, ni = _dft_mats()

    p0r = p0.reshape(E, R, C)
    p1r = p1.reshape(E, R, C)
    p2r = p2.reshape(E, C, R)
    qwr = qw.reshape(E, R, R)
    qdwr = qdw.reshape(E, R, 9).transpose(0, 2, 1).reshape(E, 9, R, 1)
    kvwr = kvw.reshape(E, 2 * R, R)
    kvdwr = kvdw.reshape(E, 2 * R, 49).transpose(0, 2, 1).reshape(E, 49, 2 * R, 1)
    lnwr = lnw.reshape(E, R, 1)
    lnbr = lnb.reshape(E, R, 1)
    powr = pow_w.reshape(E, R, R)
    pobr = pob.reshape(E, R, 1)

    def em(b, k, idx_ref):
        return (idx_ref[b * K + k], 0, 0)

    def em4(b, k, idx_ref):
        return (idx_ref[b * K + k], 0, 0, 0)

    def xb(b, k, idx_ref):
        return (b, 0, 0, 0)

    def zz(b, k, idx_ref):
        return (0, 0)

    grid_spec = pltpu.PrefetchScalarGridSpec(
        num_scalar_prefetch=1,
        grid=(B, K),
        in_specs=[
            pl.BlockSpec((1, C, H, W), xb),
            pl.BlockSpec(memory_space=pltpu.SMEM),
            pl.BlockSpec((1, R, C), em),
            pl.BlockSpec((1, R, C), em),
            pl.BlockSpec((1, C, R), em),
            pl.BlockSpec((1, R, R), em),
            pl.BlockSpec((1, 9, R, 1), em4),
            pl.BlockSpec((1, 2 * R, R), em),
            pl.BlockSpec((1, 49, 2 * R, 1), em4),
            pl.BlockSpec((1, R, 1), em),
            pl.BlockSpec((1, R, 1), em),
            pl.BlockSpec((1, R, R), em),
            pl.BlockSpec((1, R, 1), em),
            pl.BlockSpec((H, W), zz),
            pl.BlockSpec((H, W), zz),
            pl.BlockSpec((H, W), zz),
            pl.BlockSpec((H, W), zz),
        ],
        out_specs=pl.BlockSpec((1, C, H, W), xb),
    )

    out = pl.pallas_call(
        _expert_body,
        grid_spec=grid_spec,
        out_shape=jax.ShapeDtypeStruct((B, C, H, W), jnp.float32),
        compiler_params=pltpu.CompilerParams(
            dimension_semantics=("parallel", "arbitrary")),
    )(idx_flat, x, val_flat, p0r, p1r, p2r, qwr, qdwr, kvwr, kvdwr,
      lnwr, lnbr, powr, pobr, mr, mi, nr, ni)
    return out
